# baseline (device time: 28012 ns/iter reference)
import jax
import jax.numpy as jnp
from jax import lax
from jax.experimental import pallas as pl
from jax.experimental.pallas import tpu as pltpu

B = 4
S = 512
S_OUT = 256
SQ = 128
K = 512
N = 1024
CPB = 2
CR = SQ // CPB
NC = B * CPB


def kernel(O, Wo):
    O2 = O.reshape(B, S, K)

    def body(o_ref, w_ref, out_ref, w_bf, o_nb, o_my, acc, oth,
             xsend_buf, xrecv_buf, ystage, yrecv_buf,
             onb_sems, omy_sems, xsend_sems, xrecv_sems,
             ysend_sems, yrecv_sems, own_out_sems, oth_out_sems):
        my_x = lax.axis_index("x")
        my_y = lax.axis_index("y")
        ox = 1 - my_x
        oy = 1 - my_y

        barrier = pltpu.get_barrier_semaphore()
        pl.semaphore_signal(
            barrier, inc=1,
            device_id=(ox, my_y), device_id_type=pl.DeviceIdType.MESH,
        )
        pl.semaphore_signal(
            barrier, inc=1,
            device_id=(my_x, oy), device_id_type=pl.DeviceIdType.MESH,
        )
        pl.semaphore_wait(barrier, 2)

        my_q = my_x * S_OUT + my_y * SQ
        nb_q = ox * S_OUT + my_y * SQ
        loc = my_y * SQ
        sloc = oy * SQ

        nb_copies = []
        for c in range(NC):
            b, half = divmod(c, CPB)
            cp = pltpu.make_async_copy(
                o_ref.at[b, pl.ds(nb_q + half * CR, CR), :],
                o_nb.at[c],
                onb_sems.at[c],
            )
            cp.start()
            nb_copies.append(cp)
        my_copies = []
        for b in range(B):
            cp = pltpu.make_async_copy(
                o_ref.at[b, pl.ds(my_q, SQ), :],
                o_my.at[b],
                omy_sems.at[b],
            )
            cp.start()
            my_copies.append(cp)

        w_bf[...] = w_ref[...].astype(jnp.bfloat16)

        x_rdmas = []
        for c in range(NC):
            nb_copies[c].wait()
            xsend_buf[c] = jnp.dot(
                o_nb[c].astype(jnp.bfloat16),
                w_bf[...],
                preferred_element_type=jnp.float32,
            ).astype(jnp.bfloat16)
            rdma = pltpu.make_async_remote_copy(
                src_ref=xsend_buf.at[c],
                dst_ref=xrecv_buf.at[c],
                send_sem=xsend_sems.at[c],
                recv_sem=xrecv_sems.at[c],
                device_id=(ox, my_y),
                device_id_type=pl.DeviceIdType.MESH,
            )
            rdma.start()
            x_rdmas.append(rdma)

        for b in range(B):
            my_copies[b].wait()
            acc[b] = jnp.dot(
                o_my[b].astype(jnp.bfloat16),
                w_bf[...],
                preferred_element_type=jnp.float32,
            )

        y_rdmas = []
        own_out = []
        for c in range(NC):
            b, half = divmod(c, CPB)
            x_rdmas[c].wait()
            r = acc[b, pl.ds(half * CR, CR), :] + xrecv_buf[c].astype(jnp.float32)
            acc[b, pl.ds(half * CR, CR), :] = r
            ystage[c] = r.astype(jnp.bfloat16)
            yr = pltpu.make_async_remote_copy(
                src_ref=ystage.at[c],
                dst_ref=yrecv_buf.at[c],
                send_sem=ysend_sems.at[c],
                recv_sem=yrecv_sems.at[c],
                device_id=(my_x, oy),
                device_id_type=pl.DeviceIdType.MESH,
            )
            yr.start()
            y_rdmas.append(yr)
            cp = pltpu.make_async_copy(
                acc.at[b, pl.ds(half * CR, CR), :],
                out_ref.at[b, pl.ds(loc + half * CR, CR), :],
                own_out_sems.at[c],
            )
            cp.start()
            own_out.append(cp)

        oth_out = []
        for c in range(NC):
            b, half = divmod(c, CPB)
            y_rdmas[c].wait()
            oth[c] = yrecv_buf[c].astype(jnp.float32)
            cp = pltpu.make_async_copy(
                oth.at[c],
                out_ref.at[b, pl.ds(sloc + half * CR, CR), :],
                oth_out_sems.at[c],
            )
            cp.start()
            oth_out.append(cp)

        for c in range(NC):
            own_out[c].wait()
            oth_out[c].wait()

    return pl.pallas_call(
        body,
        out_shape=jax.ShapeDtypeStruct((B, S_OUT, N), jnp.float32),
        in_specs=[
            pl.BlockSpec(memory_space=pl.ANY),
            pl.BlockSpec(memory_space=pltpu.VMEM),
        ],
        out_specs=pl.BlockSpec(memory_space=pl.ANY),
        scratch_shapes=[
            pltpu.VMEM((K, N), jnp.bfloat16),
            pltpu.VMEM((NC, CR, K), jnp.float32),
            pltpu.VMEM((B, SQ, K), jnp.float32),
            pltpu.VMEM((B, SQ, N), jnp.float32),
            pltpu.VMEM((NC, CR, N), jnp.float32),
            pltpu.VMEM((NC, CR, N), jnp.bfloat16),
            pltpu.VMEM((NC, CR, N), jnp.bfloat16),
            pltpu.VMEM((NC, CR, N), jnp.bfloat16),
            pltpu.VMEM((NC, CR, N), jnp.bfloat16),
            pltpu.SemaphoreType.DMA((NC,)),
            pltpu.SemaphoreType.DMA((B,)),
            pltpu.SemaphoreType.DMA((NC,)),
            pltpu.SemaphoreType.DMA((NC,)),
            pltpu.SemaphoreType.DMA((NC,)),
            pltpu.SemaphoreType.DMA((NC,)),
            pltpu.SemaphoreType.DMA((NC,)),
            pltpu.SemaphoreType.DMA((NC,)),
        ],
        compiler_params=pltpu.CompilerParams(collective_id=0),
    )(O2, Wo)


# device time: 25093 ns/iter; 1.1163x vs baseline; 1.1163x over previous
import jax
import jax.numpy as jnp
from jax import lax
from jax.experimental import pallas as pl
from jax.experimental.pallas import tpu as pltpu

B = 4
S = 512
S_OUT = 256
SQ = 128
K = 512
N = 1024
CPB = 2
CR = SQ // CPB
NC = B * CPB


def kernel(O, Wo):
    O2 = O.reshape(B, S, K)

    def body(o_ref, w_ref, out_ref, w_bf, xsend_buf, xrecv_buf,
             ysend_buf, yrecv_buf,
             xsend_sems, xrecv_sems, ysend_sems, yrecv_sems):
        my_x = lax.axis_index("x")
        my_y = lax.axis_index("y")
        ox = 1 - my_x
        oy = 1 - my_y

        barrier = pltpu.get_barrier_semaphore()
        pl.semaphore_signal(
            barrier, inc=1,
            device_id=(ox, my_y), device_id_type=pl.DeviceIdType.MESH,
        )
        pl.semaphore_signal(
            barrier, inc=1,
            device_id=(my_x, oy), device_id_type=pl.DeviceIdType.MESH,
        )

        w_bf[...] = w_ref[...].astype(jnp.bfloat16)

        my_q = my_x * S_OUT + my_y * SQ
        nb_q = ox * S_OUT + my_y * SQ
        loc = my_y * SQ

        x_rdmas = []
        for c in range(NC):
            b, half = divmod(c, CPB)
            xsend_buf[c] = jnp.dot(
                o_ref[b, pl.ds(nb_q + half * CR, CR), :].astype(jnp.bfloat16),
                w_bf[...],
                preferred_element_type=jnp.float32,
            ).astype(jnp.bfloat16)
            if c == 0:
                pl.semaphore_wait(barrier, 2)
            rdma = pltpu.make_async_remote_copy(
                src_ref=xsend_buf.at[c],
                dst_ref=xrecv_buf.at[c],
                send_sem=xsend_sems.at[c],
                recv_sem=xrecv_sems.at[c],
                device_id=(ox, my_y),
                device_id_type=pl.DeviceIdType.MESH,
            )
            rdma.start()
            x_rdmas.append(rdma)

        for b in range(B):
            out_ref[b, pl.ds(loc, SQ), :] = jnp.dot(
                o_ref[b, pl.ds(my_q, SQ), :].astype(jnp.bfloat16),
                w_bf[...],
                preferred_element_type=jnp.float32,
            )

        y_rdmas = []
        for c in range(NC):
            b, half = divmod(c, CPB)
            row = loc + half * CR
            x_rdmas[c].wait()
            r = out_ref[b, pl.ds(row, CR), :] + xrecv_buf[c].astype(jnp.float32)
            out_ref[b, pl.ds(row, CR), :] = r
            ysend_buf[c] = r.astype(jnp.bfloat16)
            yr = pltpu.make_async_remote_copy(
                src_ref=ysend_buf.at[c],
                dst_ref=yrecv_buf.at[c],
                send_sem=ysend_sems.at[c],
                recv_sem=yrecv_sems.at[c],
                device_id=(my_x, oy),
                device_id_type=pl.DeviceIdType.MESH,
            )
            yr.start()
            y_rdmas.append(yr)

        for c in range(NC):
            b, half = divmod(c, CPB)
            row = oy * SQ + half * CR
            y_rdmas[c].wait()
            out_ref[b, pl.ds(row, CR), :] = yrecv_buf[c].astype(jnp.float32)

    return pl.pallas_call(
        body,
        out_shape=jax.ShapeDtypeStruct((B, S_OUT, N), jnp.float32),
        in_specs=[
            pl.BlockSpec(memory_space=pltpu.VMEM),
            pl.BlockSpec(memory_space=pltpu.VMEM),
        ],
        out_specs=pl.BlockSpec(memory_space=pltpu.VMEM),
        scratch_shapes=[
            pltpu.VMEM((K, N), jnp.bfloat16),
            pltpu.VMEM((NC, CR, N), jnp.bfloat16),
            pltpu.VMEM((NC, CR, N), jnp.bfloat16),
            pltpu.VMEM((NC, CR, N), jnp.bfloat16),
            pltpu.VMEM((NC, CR, N), jnp.bfloat16),
            pltpu.SemaphoreType.DMA((NC,)),
            pltpu.SemaphoreType.DMA((NC,)),
            pltpu.SemaphoreType.DMA((NC,)),
            pltpu.SemaphoreType.DMA((NC,)),
        ],
        compiler_params=pltpu.CompilerParams(collective_id=0),
    )(O2, Wo)
